# baseline (speedup 1.0000x reference)
import functools

import jax
import jax.numpy as jnp
from jax.experimental import pallas as pl
from jax.experimental.pallas import tpu as pltpu

_K = 1024
_P1_ITERS = 10
_P2_ITERS = 10
_P1_HI = 1.25
_ULP = 2.0 ** -9


def _count_lt(q, t_col):
    ones = jnp.where(q < t_col, jnp.bfloat16(1), jnp.bfloat16(0))
    colv = jnp.ones((q.shape[1], 1), jnp.bfloat16)
    return jax.lax.dot_general(
        ones, colv, (((1,), (0,)), ((), ())),
        preferred_element_type=jnp.float32)


def _bisect(q, lo0, hi0, k, iters):
    def body(_, carry):
        lo, hi = carry
        mid = 0.5 * (lo + hi)
        cnt = _count_lt(q, mid.astype(jnp.bfloat16))
        pred = cnt < k
        return jnp.where(pred, mid, lo), jnp.where(pred, hi, mid)

    return jax.lax.fori_loop(0, iters, body, (lo0, hi0))


def _bottomk_sum_kernel(x_ref, out_ref, q_ref, *, k):
    x = x_ref[...]
    rows = x.shape[0]

    q_ref[...] = x.astype(jnp.bfloat16)
    lo0 = jnp.zeros((rows, 1), jnp.float32)
    hi0 = jnp.full((rows, 1), _P1_HI, jnp.float32)
    lo, hi = _bisect(q_ref[...], lo0, hi0, k, _P1_ITERS)

    a = lo - _ULP
    width = _P1_HI * (2.0 ** -_P1_ITERS) + 2.0 * _ULP
    scale = 1.0 / width
    q_ref[...] = ((x - a) * scale).astype(jnp.bfloat16)
    lo2_0 = jnp.full((rows, 1), -0.25, jnp.float32)
    hi2_0 = jnp.full((rows, 1), 1.25, jnp.float32)
    lo2, _ = _bisect(q_ref[...], lo2_0, hi2_0, k, _P2_ITERS)
    t = a + lo2 * width

    mask = x < t
    cnt = jnp.sum(mask.astype(jnp.float32), axis=1, keepdims=True)
    ssum = jnp.sum(jnp.where(mask, x, 0.0), axis=1, keepdims=True)
    bk = ssum + (k - cnt) * t
    total = jnp.sum(bk).reshape(1, 1)

    pid = pl.program_id(0)

    @pl.when(pid == 0)
    def _():
        out_ref[...] = total

    @pl.when(pid > 0)
    def _():
        out_ref[...] += total


def _bottomk_mean(x, k, block_rows):
    rows, n = x.shape
    grid = rows // block_rows
    out = pl.pallas_call(
        functools.partial(_bottomk_sum_kernel, k=k),
        grid=(grid,),
        in_specs=[pl.BlockSpec((block_rows, n), lambda i: (i, 0))],
        out_specs=pl.BlockSpec((1, 1), lambda i: (0, 0)),
        out_shape=jax.ShapeDtypeStruct((1, 1), jnp.float32),
        scratch_shapes=[pltpu.VMEM((block_rows, n), jnp.bfloat16)],
    )(x)
    return (out[0, 0] / (rows * k)).astype(jnp.float32)


def kernel(attn):
    b, h, s, n = attn.shape
    x = attn.reshape(b * h * s, n)
    return _bottomk_mean(x, _K, block_rows=1024).reshape(())


# speedup vs baseline: 1.2747x; 1.2747x over previous
import functools

import jax
import jax.numpy as jnp
from jax.experimental import pallas as pl

_K = 1024
_ITERS = 16


def _count_lt(x, t_col):
    mask = jnp.where(x < t_col, 1.0, 0.0)
    colv = jnp.ones((x.shape[1], 1), jnp.float32)
    return jax.lax.dot_general(
        mask, colv, (((1,), (0,)), ((), ())),
        preferred_element_type=jnp.float32)


def _bottomk_sum_kernel(x_ref, out_ref, *, k, n_iters):
    x = x_ref[...]
    rows = x.shape[0]

    lo0 = jnp.zeros((rows, 1), jnp.float32)
    hi0 = jnp.ones((rows, 1), jnp.float32)

    def body(_, carry):
        lo, hi = carry
        t = 0.5 * (lo + hi)
        pred = _count_lt(x, t) < k
        return jnp.where(pred, t, lo), jnp.where(pred, hi, t)

    lo, _ = jax.lax.fori_loop(0, n_iters, body, (lo0, hi0))
    t = lo

    mask = x < t
    cnt = jnp.sum(mask.astype(jnp.float32), axis=1, keepdims=True)
    ssum = jnp.sum(jnp.where(mask, x, 0.0), axis=1, keepdims=True)
    bk = ssum + (k - cnt) * t
    total = jnp.sum(bk).reshape(1, 1)

    pid = pl.program_id(0)

    @pl.when(pid == 0)
    def _():
        out_ref[...] = total

    @pl.when(pid > 0)
    def _():
        out_ref[...] += total


def _bottomk_mean(x, k, block_rows):
    rows, n = x.shape
    grid = rows // block_rows
    out = pl.pallas_call(
        functools.partial(_bottomk_sum_kernel, k=k, n_iters=_ITERS),
        grid=(grid,),
        in_specs=[pl.BlockSpec((block_rows, n), lambda i: (i, 0))],
        out_specs=pl.BlockSpec((1, 1), lambda i: (0, 0)),
        out_shape=jax.ShapeDtypeStruct((1, 1), jnp.float32),
    )(x)
    return (out[0, 0] / (rows * k)).astype(jnp.float32)


def kernel(attn):
    b, h, s, n = attn.shape
    x = attn.reshape(b * h * s, n)
    return _bottomk_mean(x, _K, block_rows=1024).reshape(())
